# baseline (speedup 1.0000x reference)
import functools

import jax
import jax.numpy as jnp
from jax import lax
from jax.experimental import pallas as pl
from jax.experimental.pallas import tpu as pltpu
from jax.experimental.pallas import tpu_sc as plsc

HIDDEN = 128
M_BITS = 64
R_REAL = 100000
CHUNK = 2048
N_CHUNK = 49
R_PAD = CHUNK * N_CHUNK
BATCH = 1024
B_BLK = 128
B_BLK3 = 64
K = 20
KEY_SCALE = 131072.0
BIG = 1e9
SEG = 16
LANES = 128
NCAND = K * SEG
SC_CORES = 2
SC_SUBCORES = 16
SC_WORKERS = SC_CORES * SC_SUBCORES


def _k1_body(h2_ref, codes_ref, cidx_ref, m3_s):
    r = pl.program_id(1)
    d2 = lax.dot_general(h2_ref[...], codes_ref[...], (((1,), (1,)), ((), ())),
                         preferred_element_type=jnp.float32)
    lane = lax.broadcasted_iota(jnp.int32, (B_BLK, LANES), 1).astype(jnp.float32)

    def fold(last):
        m = None
        for i in range(SEG):
            if last:
                n_valid = R_REAL - ((N_CHUNK - 1) * CHUNK + i * LANES)
                if n_valid <= 0:
                    continue
            t = d2[:, i * LANES:(i + 1) * LANES] * KEY_SCALE + float(i * LANES)
            if last and n_valid < LANES:
                t = jnp.where(lane < float(n_valid), t, BIG)
            m = t if m is None else jnp.minimum(m, t)
        return m

    @pl.when(r < N_CHUNK - 1)
    def _stream():
        m3_s[r] = fold(False) + (lane + (r * CHUNK).astype(jnp.float32))

    @pl.when(r == N_CHUNK - 1)
    def _final():
        m3_s[r] = fold(True) + (lane + float((N_CHUNK - 1) * CHUNK))
        m3 = m3_s[...]
        sid3 = (lax.broadcasted_iota(jnp.int32, m3.shape, 0) * LANES
                + lax.broadcasted_iota(jnp.int32, m3.shape, 2)).astype(jnp.float32)
        cols = []
        i16 = lax.broadcasted_iota(jnp.int32, (B_BLK, SEG), 1).astype(jnp.float32)
        for _ in range(K):
            mrow = jnp.min(jnp.min(m3, axis=0), axis=1, keepdims=True)
            eq = m3 == mrow[None]
            sid = jnp.min(jnp.min(jnp.where(eq, sid3, BIG), axis=0),
                          axis=1, keepdims=True)
            m3 = jnp.where(eq, BIG, m3)
            cch = jnp.floor(sid * (1.0 / LANES))
            base = cch * float(CHUNK) + (sid - cch * float(LANES))
            cols.append(base + i16 * float(LANES))
        cidx_ref[...] = jnp.concatenate(cols, axis=1).astype(jnp.int32)


def _k1_candidates(h2, codes_pad):
    return pl.pallas_call(
        _k1_body,
        grid=(BATCH // B_BLK, N_CHUNK),
        in_specs=[
            pl.BlockSpec((B_BLK, M_BITS), lambda b, r: (b, 0)),
            pl.BlockSpec((CHUNK, M_BITS), lambda b, r: (r, 0)),
        ],
        out_specs=pl.BlockSpec((B_BLK, NCAND), lambda b, r: (b, 0)),
        out_shape=jax.ShapeDtypeStruct((BATCH, NCAND), jnp.int32),
        scratch_shapes=[pltpu.VMEM((N_CHUNK, B_BLK, LANES), jnp.float32)],
        compiler_params=pltpu.CompilerParams(
            dimension_semantics=("arbitrary", "arbitrary")),
    )(h2, codes_pad)


def _sc_gather_rows(table, idx, rows_per_worker, chunk=128, nbuf=4):
    n, d = idx.shape[0], table.shape[1]
    n_iter = rows_per_worker // (chunk * nbuf)
    assert rows_per_worker % (chunk * nbuf) == 0
    mesh = plsc.VectorSubcoreMesh(core_axis_name="c", subcore_axis_name="s")

    @functools.partial(
        pl.kernel, mesh=mesh,
        out_type=jax.ShapeDtypeStruct((n, d), table.dtype),
        scratch_types=[
            pltpu.VMEM((nbuf, chunk), jnp.int32),
            pltpu.VMEM((nbuf, chunk, d), table.dtype),
            pltpu.SemaphoreType.DMA,
            pltpu.SemaphoreType.DMA,
            pltpu.SemaphoreType.DMA,
        ],
    )
    def gather_kernel(table_hbm, idx_hbm, out_hbm, idx_v, rows_v,
                      sem_i, sem_g, sem_o):
        wid = lax.axis_index("s") * SC_CORES + lax.axis_index("c")
        base = wid * rows_per_worker

        def body(g, carry):
            off = base + g * (chunk * nbuf)
            ic = [pltpu.async_copy(
                idx_hbm.at[pl.ds(off + b * chunk, chunk)],
                idx_v.at[b], sem_i) for b in range(nbuf)]
            for c in ic:
                c.wait()
            gc = [pltpu.async_copy(table_hbm.at[idx_v.at[b]],
                                   rows_v.at[b], sem_g) for b in range(nbuf)]
            for c in gc:
                c.wait()
            oc = [pltpu.async_copy(
                rows_v.at[b], out_hbm.at[pl.ds(off + b * chunk, chunk)],
                sem_o) for b in range(nbuf)]
            for c in oc:
                c.wait()
            return carry

        lax.fori_loop(0, n_iter, body, 0)

    return gather_kernel(table, idx)


def _k3_body(cand_ref, h2_ref, cidx_ref, sim_ref, lab_ref):
    c3 = cand_ref[:, :, 0:M_BITS].astype(jnp.float32)
    lab = cand_ref[:, :, M_BITS]
    h2 = h2_ref[...].astype(jnp.float32)
    d2 = jnp.sum(c3 * h2.reshape(B_BLK3, 1, M_BITS), axis=2)
    hsum = (float(M_BITS) - jnp.sum(h2, axis=1, keepdims=True)) * 0.5
    ham = hsum + d2
    colf = cidx_ref[...].astype(jnp.float32)
    key0 = jnp.where(colf < float(R_REAL), ham * KEY_SCALE + colf, BIG)
    cand = key0
    keys, labs = [], []
    for _ in range(K):
        kmin = jnp.min(cand, axis=1, keepdims=True)
        cand = jnp.where(cand == kmin, BIG, cand)
        keys.append(kmin)
        labs.append(jnp.sum(jnp.where(key0 == kmin, lab, 0),
                            axis=1, keepdims=True))
    kmat = jnp.concatenate(keys, axis=1)
    hamv = jnp.floor(kmat * (1.0 / KEY_SCALE))
    w = hamv * (-1.0 / float(M_BITS))
    e = jnp.exp(w - jnp.max(w, axis=1, keepdims=True))
    sim_ref[...] = e / jnp.sum(e, axis=1, keepdims=True)
    lab_ref[...] = jnp.concatenate(labs, axis=1)


def _k3_rescore(cand_codes, h2, cand_idx):
    return pl.pallas_call(
        _k3_body,
        grid=(BATCH // B_BLK3,),
        in_specs=[
            pl.BlockSpec((B_BLK3, NCAND, 2 * M_BITS), lambda b: (b, 0, 0)),
            pl.BlockSpec((B_BLK3, M_BITS), lambda b: (b, 0)),
            pl.BlockSpec((B_BLK3, NCAND), lambda b: (b, 0)),
        ],
        out_specs=[
            pl.BlockSpec((B_BLK3, K), lambda b: (b, 0)),
            pl.BlockSpec((B_BLK3, K), lambda b: (b, 0)),
        ],
        out_shape=[
            jax.ShapeDtypeStruct((BATCH, K), jnp.float32),
            jax.ShapeDtypeStruct((BATCH, K), jnp.int32),
        ],
    )(cand_codes, h2, cand_idx)


def _pad_codes(sess_codes, dtype, width=M_BITS):
    return jnp.pad(sess_codes,
                   ((0, R_PAD - R_REAL), (0, width - M_BITS))).astype(dtype)


def _gather_table(sess_codes, his_labels):
    return jnp.concatenate([
        _pad_codes(sess_codes, jnp.int32),
        jnp.pad(his_labels, (0, R_PAD - R_REAL)).reshape(R_PAD, 1),
        jnp.zeros((R_PAD, M_BITS - 1), jnp.int32),
    ], axis=1)


def kernel(target_sess, sess_codes, his_labels, sess_hash_matrix, topk):
    proj = jnp.matmul(target_sess, sess_hash_matrix)
    h2 = (1.0 - 2.0 * jnp.where(proj > 0, 1.0, 0.0)).astype(jnp.bfloat16)
    cand_idx = _k1_candidates(h2, _pad_codes(sess_codes, jnp.bfloat16))
    cand_rows = _sc_gather_rows(
        _gather_table(sess_codes, his_labels),
        cand_idx.reshape(-1),
        rows_per_worker=BATCH * NCAND // SC_WORKERS,
    ).reshape(BATCH, NCAND, 2 * M_BITS)
    sim, labels = _k3_rescore(cand_rows, h2, cand_idx)
    return sim, labels


# speedup vs baseline: 1.1077x; 1.1077x over previous
import functools

import jax
import jax.numpy as jnp
from jax import lax
from jax.experimental import pallas as pl
from jax.experimental.pallas import tpu as pltpu
from jax.experimental.pallas import tpu_sc as plsc

HIDDEN = 128
M_BITS = 64
R_REAL = 100000
CHUNK = 2048
N_CHUNK = 49
R_PAD = CHUNK * N_CHUNK
BATCH = 1024
B_BLK = 128
B_BLK3 = 64
K = 20
KEY_SCALE = 131072.0
BIG = 1e9
SEG = 16
LANES = 128
NCAND = K * SEG
SC_CORES = 2
SC_SUBCORES = 16
SC_WORKERS = SC_CORES * SC_SUBCORES


def _k1_body(h2_ref, codes_ref, cidx_ref, m3_s):
    r = pl.program_id(1)
    d2 = lax.dot_general(h2_ref[...], codes_ref[...], (((1,), (1,)), ((), ())),
                         preferred_element_type=jnp.float32)
    lane = lax.broadcasted_iota(jnp.int32, (B_BLK, LANES), 1).astype(jnp.float32)

    def fold(last):
        m = None
        for i in range(SEG):
            if last:
                n_valid = R_REAL - ((N_CHUNK - 1) * CHUNK + i * LANES)
                if n_valid <= 0:
                    continue
            t = d2[:, i * LANES:(i + 1) * LANES] * KEY_SCALE + float(i * LANES)
            if last and n_valid < LANES:
                t = jnp.where(lane < float(n_valid), t, BIG)
            m = t if m is None else jnp.minimum(m, t)
        return m

    @pl.when(r < N_CHUNK - 1)
    def _stream():
        m3_s[r] = fold(False) + (lane + (r * CHUNK).astype(jnp.float32))

    @pl.when(r == N_CHUNK - 1)
    def _final():
        m3_s[r] = fold(True) + (lane + float((N_CHUNK - 1) * CHUNK))
        m3 = m3_s[...]
        sid3 = (lax.broadcasted_iota(jnp.int32, m3.shape, 0) * LANES
                + lax.broadcasted_iota(jnp.int32, m3.shape, 2)).astype(jnp.float32)
        cols = []
        i16 = lax.broadcasted_iota(jnp.int32, (B_BLK, SEG), 1).astype(jnp.float32)
        for _ in range(K):
            mrow = jnp.min(jnp.min(m3, axis=0), axis=1, keepdims=True)
            eq = m3 == mrow[None]
            sid = jnp.min(jnp.min(jnp.where(eq, sid3, BIG), axis=0),
                          axis=1, keepdims=True)
            m3 = jnp.where(eq, BIG, m3)
            cch = jnp.floor(sid * (1.0 / LANES))
            base = cch * float(CHUNK) + (sid - cch * float(LANES))
            cols.append(base + i16 * float(LANES))
        cidx_ref[...] = jnp.concatenate(cols, axis=1).astype(jnp.int32)


def _k1_candidates(h2, codes_pad):
    batch = h2.shape[0]
    return pl.pallas_call(
        _k1_body,
        grid=(batch // B_BLK, N_CHUNK),
        in_specs=[
            pl.BlockSpec((B_BLK, M_BITS), lambda b, r: (b, 0)),
            pl.BlockSpec((CHUNK, M_BITS), lambda b, r: (r, 0)),
        ],
        out_specs=pl.BlockSpec((B_BLK, NCAND), lambda b, r: (b, 0)),
        out_shape=jax.ShapeDtypeStruct((batch, NCAND), jnp.int32),
        scratch_shapes=[pltpu.VMEM((N_CHUNK, B_BLK, LANES), jnp.float32)],
        compiler_params=pltpu.CompilerParams(
            dimension_semantics=("arbitrary", "arbitrary")),
    )(h2, codes_pad)


def _sc_gather_rows(table, idx, rows_per_worker, chunk=128, nbuf=4):
    n, d = idx.shape[0], table.shape[1]
    n_iter = rows_per_worker // (chunk * nbuf)
    assert rows_per_worker % (chunk * nbuf) == 0
    mesh = plsc.VectorSubcoreMesh(core_axis_name="c", subcore_axis_name="s")

    @functools.partial(
        pl.kernel, mesh=mesh,
        out_type=jax.ShapeDtypeStruct((n, d), table.dtype),
        scratch_types=[
            pltpu.VMEM((nbuf, chunk), jnp.int32),
            pltpu.VMEM((nbuf, chunk, d), table.dtype),
            pltpu.SemaphoreType.DMA,
            pltpu.SemaphoreType.DMA,
            pltpu.SemaphoreType.DMA,
        ],
    )
    def gather_kernel(table_hbm, idx_hbm, out_hbm, idx_v, rows_v,
                      sem_i, sem_g, sem_o):
        wid = lax.axis_index("s") * SC_CORES + lax.axis_index("c")
        base = wid * rows_per_worker

        def body(g, carry):
            off = base + g * (chunk * nbuf)
            ic = [pltpu.async_copy(
                idx_hbm.at[pl.ds(off + b * chunk, chunk)],
                idx_v.at[b], sem_i) for b in range(nbuf)]
            for c in ic:
                c.wait()
            gc = [pltpu.async_copy(table_hbm.at[idx_v.at[b]],
                                   rows_v.at[b], sem_g) for b in range(nbuf)]
            for c in gc:
                c.wait()
            oc = [pltpu.async_copy(
                rows_v.at[b], out_hbm.at[pl.ds(off + b * chunk, chunk)],
                sem_o) for b in range(nbuf)]
            for c in oc:
                c.wait()
            return carry

        lax.fori_loop(0, n_iter, body, 0)

    return gather_kernel(table, idx)


def _k3_body(cand_ref, h2_ref, cidx_ref, sim_ref, lab_ref):
    c3 = cand_ref[:, :, 0:M_BITS].astype(jnp.float32)
    lab = cand_ref[:, :, M_BITS]
    h2 = h2_ref[...].astype(jnp.float32)
    d2 = jnp.sum(c3 * h2.reshape(B_BLK3, 1, M_BITS), axis=2)
    hsum = (float(M_BITS) - jnp.sum(h2, axis=1, keepdims=True)) * 0.5
    ham = hsum + d2
    colf = cidx_ref[...].astype(jnp.float32)
    key0 = jnp.where(colf < float(R_REAL), ham * KEY_SCALE + colf, BIG)
    cand = key0
    keys, labs = [], []
    for _ in range(K):
        kmin = jnp.min(cand, axis=1, keepdims=True)
        cand = jnp.where(cand == kmin, BIG, cand)
        keys.append(kmin)
        labs.append(jnp.sum(jnp.where(key0 == kmin, lab, 0),
                            axis=1, keepdims=True))
    kmat = jnp.concatenate(keys, axis=1)
    hamv = jnp.floor(kmat * (1.0 / KEY_SCALE))
    w = hamv * (-1.0 / float(M_BITS))
    e = jnp.exp(w - jnp.max(w, axis=1, keepdims=True))
    sim_ref[...] = e / jnp.sum(e, axis=1, keepdims=True)
    lab_ref[...] = jnp.concatenate(labs, axis=1)


def _k3_rescore(cand_codes, h2, cand_idx):
    batch = h2.shape[0]
    return pl.pallas_call(
        _k3_body,
        grid=(batch // B_BLK3,),
        in_specs=[
            pl.BlockSpec((B_BLK3, NCAND, 2 * M_BITS), lambda b: (b, 0, 0)),
            pl.BlockSpec((B_BLK3, M_BITS), lambda b: (b, 0)),
            pl.BlockSpec((B_BLK3, NCAND), lambda b: (b, 0)),
        ],
        out_specs=[
            pl.BlockSpec((B_BLK3, K), lambda b: (b, 0)),
            pl.BlockSpec((B_BLK3, K), lambda b: (b, 0)),
        ],
        out_shape=[
            jax.ShapeDtypeStruct((batch, K), jnp.float32),
            jax.ShapeDtypeStruct((batch, K), jnp.int32),
        ],
    )(cand_codes, h2, cand_idx)


def _pad_codes(sess_codes, dtype, width=M_BITS):
    return jnp.pad(sess_codes,
                   ((0, R_PAD - R_REAL), (0, width - M_BITS))).astype(dtype)


def _gather_table(sess_codes, his_labels):
    return jnp.concatenate([
        _pad_codes(sess_codes, jnp.int32),
        jnp.pad(his_labels, (0, R_PAD - R_REAL)).reshape(R_PAD, 1),
        jnp.zeros((R_PAD, M_BITS - 1), jnp.int32),
    ], axis=1)


def kernel(target_sess, sess_codes, his_labels, sess_hash_matrix, topk):
    proj = jnp.matmul(target_sess, sess_hash_matrix)
    h2 = (1.0 - 2.0 * jnp.where(proj > 0, 1.0, 0.0)).astype(jnp.bfloat16)
    codes_bf = _pad_codes(sess_codes, jnp.bfloat16)
    table = _gather_table(sess_codes, his_labels)
    half = BATCH // 2
    sims, labss = [], []
    for p in range(2):
        h2p = h2[p * half:(p + 1) * half]
        cand_idx = _k1_candidates(h2p, codes_bf)
        cand_rows = _sc_gather_rows(
            table, cand_idx.reshape(-1),
            rows_per_worker=half * NCAND // SC_WORKERS,
        ).reshape(half, NCAND, 2 * M_BITS)
        sim, labels = _k3_rescore(cand_rows, h2p, cand_idx)
        sims.append(sim)
        labss.append(labels)
    return (jnp.concatenate(sims, axis=0), jnp.concatenate(labss, axis=0))
